# Initial kernel scaffold; baseline (speedup 1.0000x reference)
import jax
import jax.numpy as jnp
from jax.experimental import pallas as pl


def kernel(x, hyperedge_index, W, bias):
    raise NotImplementedError("write your pallas kernel here")


# speedup vs baseline: 8.4871x; 8.4871x over previous
import functools

import jax
import jax.numpy as jnp
from jax import lax
from jax.experimental import pallas as pl
from jax.experimental.pallas import tpu as pltpu
from jax.experimental.pallas import tpu_sc as plsc

N_NODES = 10000
NNZ = 320000
CH = 128
NC, NS, LANES = 2, 16, 16
NW = NC * NS
NP = 10240
CHUNK = 128
NNZ_P = 327680
ROWS_PER_TILE = NP // NS
ZCOPIES = ROWS_PER_TILE // CHUNK
HIST_CHUNKS = NNZ_P // NS // CHUNK
SEG_CHUNKS = NNZ_P // NW // CHUNK
DUMMY = N_NODES

@functools.lru_cache(maxsize=None)
def _mesh():
    return plsc.VectorSubcoreMesh(core_axis_name="c", subcore_axis_name="s",
                                  num_cores=NC, num_subcores=NS)

def _fill_vmem_constants(bufs_and_rows):
    def step(i, _):
        for buf, rowvec in bufs_and_rows:
            w = rowvec.shape[0]
            for j in range(buf.shape[1] // w):
                buf[i, pl.ds(j * w, w)] = rowvec
        return 0
    lax.fori_loop(0, CHUNK, step, 0)


def _seg_body(with_hist, gdim, sdim, idx2, table, *refs):
    if with_hist:
        (hist_out, p_out, zbuf, z1, on1, hidx, gidx, sidx, rows,
         hist_sp, acc_sp, sem) = refs
    else:
        (p_out, zbuf, gidx, sidx, rows, acc_sp, sem) = refs

    c = lax.axis_index("c")
    s = lax.axis_index("s")
    base = s * ROWS_PER_TILE

    zeros16 = jnp.zeros((LANES,), jnp.float32)
    _fill_vmem_constants([(zbuf, zeros16)])
    if with_hist:
        ones16 = jnp.full((LANES,), 1.0, jnp.float32)
        for j in range(CHUNK // LANES):
            z1[pl.ds(j * LANES, LANES)] = zeros16
            on1[pl.ds(j * LANES, LANES)] = ones16

    for k in range(ZCOPIES):
        r0 = base + k * CHUNK
        pltpu.sync_copy(zbuf, acc_sp.at[pl.ds(r0, CHUNK)])
        if with_hist:
            pltpu.sync_copy(z1, hist_sp.at[pl.ds(r0, CHUNK)])
    plsc.subcore_barrier()

    if with_hist:
        hbase = s * (NNZ_P // NS)
        def hist_step(t, _):
            pltpu.sync_copy(idx2.at[c, pl.ds(hbase + t * CHUNK, CHUNK)], hidx)
            pltpu.sync_copy(on1, hist_sp.at[hidx], add=True)
            return 0
        lax.fori_loop(0, HIST_CHUNKS, hist_step, 0)
        plsc.subcore_barrier()
        for k in range(ZCOPIES):
            r0 = base + k * CHUNK
            pltpu.sync_copy(hist_sp.at[pl.ds(r0, CHUNK)],
                            hist_out.at[c, pl.ds(r0, CHUNK)])

    ebase = (c * NS + s) * (NNZ_P // NW)
    def seg_step(t, _):
        off = ebase + t * CHUNK
        pltpu.sync_copy(idx2.at[gdim, pl.ds(off, CHUNK)], gidx)
        pltpu.sync_copy(idx2.at[sdim, pl.ds(off, CHUNK)], sidx)
        pltpu.async_copy(table.at[gidx], rows, sem).wait()
        pltpu.sync_copy(rows, acc_sp.at[sidx], add=True)
        return 0
    lax.fori_loop(0, SEG_CHUNKS, seg_step, 0)
    plsc.subcore_barrier()
    for k in range(ZCOPIES):
        r0 = base + k * CHUNK
        pltpu.sync_copy(acc_sp.at[pl.ds(r0, CHUNK)],
                        p_out.at[c, pl.ds(r0, CHUNK)])


def _sc_pass_a_body(idx2, table, hist_out, p_out, *scratch):
    _seg_body(True, 0, 1, idx2, table, hist_out, p_out, *scratch)


def _sc_pass_b_body(idx2, table, p_out, *scratch):
    _seg_body(False, 1, 0, idx2, table, p_out, *scratch)


@functools.lru_cache(maxsize=None)
def _sc_pass_a():
    return pl.kernel(
        _sc_pass_a_body,
        out_type=(jax.ShapeDtypeStruct((NC, NP), jnp.float32),
                  jax.ShapeDtypeStruct((NC, NP, CH), jnp.float32)),
        mesh=_mesh(),
        scratch_types=(
            pltpu.VMEM((CHUNK, CH), jnp.float32),
            pltpu.VMEM((CHUNK,), jnp.float32),
            pltpu.VMEM((CHUNK,), jnp.float32),
            pltpu.VMEM((CHUNK,), jnp.int32),
            pltpu.VMEM((CHUNK,), jnp.int32),
            pltpu.VMEM((CHUNK,), jnp.int32),
            pltpu.VMEM((CHUNK, CH), jnp.float32),
            pltpu.VMEM_SHARED((NP,), jnp.float32),
            pltpu.VMEM_SHARED((NP, CH), jnp.float32),
            pltpu.SemaphoreType.DMA,
        ),
    )


@functools.lru_cache(maxsize=None)
def _sc_pass_b():
    return pl.kernel(
        _sc_pass_b_body,
        out_type=jax.ShapeDtypeStruct((NC, NP, CH), jnp.float32),
        mesh=_mesh(),
        scratch_types=(
            pltpu.VMEM((CHUNK, CH), jnp.float32),
            pltpu.VMEM((CHUNK,), jnp.int32),
            pltpu.VMEM((CHUNK,), jnp.int32),
            pltpu.VMEM((CHUNK, CH), jnp.float32),
            pltpu.VMEM_SHARED((NP, CH), jnp.float32),
            pltpu.SemaphoreType.DMA,
        ),
    )

_BLK = 1024


def _combine_body(hist_ref, p0_ref, p1_ref, o_ref):
    b = hist_ref[...]
    scale = jnp.where(b > 0, 1.0 / b, 0.0)
    o_ref[...] = scale * (p0_ref[...] + p1_ref[...])


_combine = pl.pallas_call(
    _combine_body,
    grid=(NP // _BLK,),
    in_specs=[
        pl.BlockSpec((_BLK, 1), lambda i: (i, 0)),
        pl.BlockSpec((_BLK, CH), lambda i: (i, 0)),
        pl.BlockSpec((_BLK, CH), lambda i: (i, 0)),
    ],
    out_specs=pl.BlockSpec((_BLK, CH), lambda i: (i, 0)),
    out_shape=jax.ShapeDtypeStruct((NP, CH), jnp.float32),
)


def _final_body(hist_ref, q0_ref, q1_ref, w_ref, b_ref, o_ref):
    d = hist_ref[...]
    scale = jnp.where(d > 0, 1.0 / d, 0.0)
    y = scale * (q0_ref[...] + q1_ref[...])
    o_ref[...] = lax.dot_general(
        y, w_ref[...], (((1,), (1,)), ((), ())),
        preferred_element_type=jnp.float32) + b_ref[...]


_final = pl.pallas_call(
    _final_body,
    grid=(NP // _BLK,),
    in_specs=[
        pl.BlockSpec((_BLK, 1), lambda i: (i, 0)),
        pl.BlockSpec((_BLK, CH), lambda i: (i, 0)),
        pl.BlockSpec((_BLK, CH), lambda i: (i, 0)),
        pl.BlockSpec((CH, CH), lambda i: (0, 0)),
        pl.BlockSpec((1, CH), lambda i: (0, 0)),
    ],
    out_specs=pl.BlockSpec((_BLK, CH), lambda i: (i, 0)),
    out_shape=jax.ShapeDtypeStruct((NP, CH), jnp.float32),
)


def kernel(x, hyperedge_index, W, bias):
    fill = jnp.full((NNZ_P - NNZ,), DUMMY, jnp.int32)
    idx2 = jnp.stack([jnp.concatenate([hyperedge_index[0], fill]),
                      jnp.concatenate([hyperedge_index[1], fill])])
    x_pad = jnp.concatenate(
        [x, jnp.zeros((NP - N_NODES, CH), jnp.float32)])

    hist, p = _sc_pass_a()(idx2, x_pad)
    e_pad = _combine(hist[1].reshape(NP, 1), p[0], p[1])
    q = _sc_pass_b()(idx2, e_pad)
    out_pad = _final(hist[0].reshape(NP, 1), q[0], q[1], W,
                     bias.reshape(1, CH))
    return out_pad[:N_NODES]


# speedup vs baseline: 9.6494x; 1.1369x over previous
import functools

import jax
import jax.numpy as jnp
from jax import lax
from jax.experimental import pallas as pl
from jax.experimental.pallas import tpu as pltpu
from jax.experimental.pallas import tpu_sc as plsc

N_NODES = 10000
NNZ = 320000
CH = 128
NC, NS, LANES = 2, 16, 16
NW = NC * NS
NP = 10240
CHUNK = 128
NNZ_P = 327680
EPT = NNZ_P // NW
NCHUNK = EPT // CHUNK
ROWS_PER_TILE = NP // NS
ZROWS = 64
ZC = ROWS_PER_TILE // ZROWS
DUMMY = N_NODES


@functools.lru_cache(maxsize=None)
def _mesh():
    return plsc.VectorSubcoreMesh(core_axis_name="c", subcore_axis_name="s",
                                  num_cores=NC, num_subcores=NS)


def _seg_body(with_hist, gdim, sdim, idx2, table, *refs):
    if with_hist:
        (hr_out, hc_out, p_out, zbuf, z640, ones, gb0, gb1, sb0, sb1,
         rows0, rows1, hr_sp, hc_sp, acc_sp,
         semi0, semi1, sem0, sem1) = refs
    else:
        (p_out, zbuf, gb0, gb1, sb0, sb1, rows0, rows1,
         acc_sp, semi0, semi1, sem0, sem1) = refs

    c = lax.axis_index("c")
    s = lax.axis_index("s")
    wid = c * NS + s
    base = s * ROWS_PER_TILE

    def previdx(t, gb, sb, semi):
        pltpu.async_copy(idx2.at[gdim, wid, pl.ds(t * CHUNK, CHUNK)],
                         gb, semi)
        pltpu.async_copy(idx2.at[sdim, wid, pl.ds(t * CHUNK, CHUNK)],
                         sb, semi)

    def waitidx(gb, sb, semi):
        pltpu.make_async_copy(idx2.at[gdim, wid, pl.ds(0, CHUNK)],
                              gb, semi).wait()
        pltpu.make_async_copy(idx2.at[sdim, wid, pl.ds(0, CHUNK)],
                              sb, semi).wait()

    previdx(0, gb0, sb0, semi0)
    previdx(1, gb1, sb1, semi1)

    zeros16 = jnp.zeros((LANES,), jnp.float32)
    def zrow(i, _):
        for j in range(CH // LANES):
            zbuf[i, pl.ds(j * LANES, LANES)] = zeros16
        return 0
    lax.fori_loop(0, ZROWS, zrow, 0)
    if with_hist:
        ones16 = jnp.full((LANES,), 1.0, jnp.float32)
        for j in range(CHUNK // LANES):
            ones[pl.ds(j * LANES, LANES)] = ones16
        for j in range(ROWS_PER_TILE // LANES):
            z640[pl.ds(j * LANES, LANES)] = zeros16

    for k in range(ZC):
        pltpu.sync_copy(zbuf, acc_sp.at[pl.ds(base + k * ZROWS, ZROWS)])
    if with_hist:
        pltpu.sync_copy(z640, hr_sp.at[pl.ds(base, ROWS_PER_TILE)])
        pltpu.sync_copy(z640, hc_sp.at[pl.ds(base, ROWS_PER_TILE)])
    plsc.subcore_barrier()

    def gather(gb, rows, sem):
        pltpu.async_copy(table.at[gb], rows, sem)

    def drain(rows, sem):
        pltpu.make_async_copy(table.at[pl.ds(0, CHUNK)], rows, sem).wait()

    def finish(rows, gb, sb):
        pltpu.sync_copy(rows, acc_sp.at[sb], add=True)
        if with_hist:
            pltpu.sync_copy(ones, hr_sp.at[gb], add=True)
            pltpu.sync_copy(ones, hc_sp.at[sb], add=True)

    waitidx(gb0, sb0, semi0)
    gather(gb0, rows0, sem0)
    def pair(j, _):
        t0 = 2 * j
        waitidx(gb1, sb1, semi1)
        gather(gb1, rows1, sem1)
        drain(rows0, sem0)
        finish(rows0, gb0, sb0)
        @pl.when(t0 + 2 < NCHUNK)
        def _():
            previdx(t0 + 2, gb0, sb0, semi0)
        drain(rows1, sem1)
        finish(rows1, gb1, sb1)
        @pl.when(t0 + 2 < NCHUNK)
        def _():
            waitidx(gb0, sb0, semi0)
            gather(gb0, rows0, sem0)
        @pl.when(t0 + 3 < NCHUNK)
        def _():
            previdx(t0 + 3, gb1, sb1, semi1)
        return 0
    lax.fori_loop(0, NCHUNK // 2, pair, 0)
    plsc.subcore_barrier()

    pltpu.sync_copy(acc_sp.at[pl.ds(base, ROWS_PER_TILE)],
                    p_out.at[c, pl.ds(base, ROWS_PER_TILE)])
    if with_hist:
        pltpu.sync_copy(hr_sp.at[pl.ds(base, ROWS_PER_TILE)],
                        hr_out.at[c, pl.ds(base, ROWS_PER_TILE)])
        pltpu.sync_copy(hc_sp.at[pl.ds(base, ROWS_PER_TILE)],
                        hc_out.at[c, pl.ds(base, ROWS_PER_TILE)])


def _sc_pass_a_body(idx2, table, hr_out, hc_out, p_out, *scratch):
    _seg_body(True, 0, 1, idx2, table, hr_out, hc_out, p_out, *scratch)


def _sc_pass_b_body(idx2, table, p_out, *scratch):
    _seg_body(False, 1, 0, idx2, table, p_out, *scratch)


@functools.lru_cache(maxsize=None)
def _sc_pass_a():
    return pl.kernel(
        _sc_pass_a_body,
        out_type=(jax.ShapeDtypeStruct((NC, NP), jnp.float32),
                  jax.ShapeDtypeStruct((NC, NP), jnp.float32),
                  jax.ShapeDtypeStruct((NC, NP, CH), jnp.float32)),
        mesh=_mesh(),
        scratch_types=(
            pltpu.VMEM((ZROWS, CH), jnp.float32),
            pltpu.VMEM((ROWS_PER_TILE,), jnp.float32),
            pltpu.VMEM((CHUNK,), jnp.float32),
            pltpu.VMEM((CHUNK,), jnp.int32),
            pltpu.VMEM((CHUNK,), jnp.int32),
            pltpu.VMEM((CHUNK,), jnp.int32),
            pltpu.VMEM((CHUNK,), jnp.int32),
            pltpu.VMEM((CHUNK, CH), jnp.float32),
            pltpu.VMEM((CHUNK, CH), jnp.float32),
            pltpu.VMEM_SHARED((NP,), jnp.float32),
            pltpu.VMEM_SHARED((NP,), jnp.float32),
            pltpu.VMEM_SHARED((NP, CH), jnp.float32),
            pltpu.SemaphoreType.DMA,
            pltpu.SemaphoreType.DMA,
            pltpu.SemaphoreType.DMA,
            pltpu.SemaphoreType.DMA,
        ),
    )


@functools.lru_cache(maxsize=None)
def _sc_pass_b():
    return pl.kernel(
        _sc_pass_b_body,
        out_type=jax.ShapeDtypeStruct((NC, NP, CH), jnp.float32),
        mesh=_mesh(),
        scratch_types=(
            pltpu.VMEM((ZROWS, CH), jnp.float32),
            pltpu.VMEM((CHUNK,), jnp.int32),
            pltpu.VMEM((CHUNK,), jnp.int32),
            pltpu.VMEM((CHUNK,), jnp.int32),
            pltpu.VMEM((CHUNK,), jnp.int32),
            pltpu.VMEM((CHUNK, CH), jnp.float32),
            pltpu.VMEM((CHUNK, CH), jnp.float32),
            pltpu.VMEM_SHARED((NP, CH), jnp.float32),
            pltpu.SemaphoreType.DMA,
            pltpu.SemaphoreType.DMA,
            pltpu.SemaphoreType.DMA,
            pltpu.SemaphoreType.DMA,
        ),
    )


_BLK = 1024


def _combine_body(h0_ref, h1_ref, p0_ref, p1_ref, o_ref):
    b = h0_ref[...] + h1_ref[...]
    scale = jnp.where(b > 0, 1.0 / b, 0.0)
    o_ref[...] = scale * (p0_ref[...] + p1_ref[...])


_combine = pl.pallas_call(
    _combine_body,
    grid=(NP // _BLK,),
    in_specs=[
        pl.BlockSpec((_BLK, 1), lambda i: (i, 0)),
        pl.BlockSpec((_BLK, 1), lambda i: (i, 0)),
        pl.BlockSpec((_BLK, CH), lambda i: (i, 0)),
        pl.BlockSpec((_BLK, CH), lambda i: (i, 0)),
    ],
    out_specs=pl.BlockSpec((_BLK, CH), lambda i: (i, 0)),
    out_shape=jax.ShapeDtypeStruct((NP, CH), jnp.float32),
)


def _final_body(h0_ref, h1_ref, q0_ref, q1_ref, w_ref, b_ref, o_ref):
    d = h0_ref[...] + h1_ref[...]
    scale = jnp.where(d > 0, 1.0 / d, 0.0)
    y = scale * (q0_ref[...] + q1_ref[...])
    o_ref[...] = lax.dot_general(
        y, w_ref[...], (((1,), (1,)), ((), ())),
        preferred_element_type=jnp.float32) + b_ref[...]


_final = pl.pallas_call(
    _final_body,
    grid=(NP // _BLK,),
    in_specs=[
        pl.BlockSpec((_BLK, 1), lambda i: (i, 0)),
        pl.BlockSpec((_BLK, 1), lambda i: (i, 0)),
        pl.BlockSpec((_BLK, CH), lambda i: (i, 0)),
        pl.BlockSpec((_BLK, CH), lambda i: (i, 0)),
        pl.BlockSpec((CH, CH), lambda i: (0, 0)),
        pl.BlockSpec((1, CH), lambda i: (0, 0)),
    ],
    out_specs=pl.BlockSpec((_BLK, CH), lambda i: (i, 0)),
    out_shape=jax.ShapeDtypeStruct((NP, CH), jnp.float32),
)


def kernel(x, hyperedge_index, W, bias):
    fill = jnp.full((NNZ_P - NNZ,), DUMMY, jnp.int32)
    idx2 = jnp.stack([jnp.concatenate([hyperedge_index[0], fill]),
                      jnp.concatenate([hyperedge_index[1], fill])])
    idx2 = idx2.reshape(2, NW, EPT)
    x_pad = jnp.concatenate(
        [x, jnp.zeros((NP - N_NODES, CH), jnp.float32)])

    hr, hc, p = _sc_pass_a()(idx2, x_pad)
    e_pad = _combine(hc[0].reshape(NP, 1), hc[1].reshape(NP, 1), p[0], p[1])
    q = _sc_pass_b()(idx2, e_pad)
    out_pad = _final(hr[0].reshape(NP, 1), hr[1].reshape(NP, 1),
                     q[0], q[1], W, bias.reshape(1, CH))
    return out_pad[:N_NODES]


# speedup vs baseline: 28.3587x; 2.9389x over previous
import functools

import jax
import jax.numpy as jnp
from jax import lax
from jax.experimental import pallas as pl
from jax.experimental.pallas import tpu as pltpu
from jax.experimental.pallas import tpu_sc as plsc

N_NODES = 10000
NNZ = 320000
CH = 128
NC, NS, LANES = 2, 16, 16
NW = NC * NS
NP = 10240
CHUNK = 128
NNZ_P = 327680
EPT = NNZ_P // NW
NCHUNK = EPT // CHUNK
ROWS_PER_TILE = NP // NS
ZROWS = 64
ZC = ROWS_PER_TILE // ZROWS
DUMMY = N_NODES


@functools.lru_cache(maxsize=None)
def _mesh():
    return plsc.VectorSubcoreMesh(core_axis_name="c", subcore_axis_name="s",
                                  num_cores=NC, num_subcores=NS)


def _seg_body(with_hist, gdim, sdim, idx2, table, *refs):
    if with_hist:
        (hr_out, hc_out, p_out, zbuf, z640, ones, gb0, gb1, sb0, sb1,
         rows0, rows1, hr_sp, hc_sp, acc_sp,
         semi0, semi1, sem0, sem1) = refs
    else:
        (p_out, zbuf, gb0, gb1, sb0, sb1, rows0, rows1,
         acc_sp, semi0, semi1, sem0, sem1) = refs

    c = lax.axis_index("c")
    s = lax.axis_index("s")
    wid = c * NS + s
    base = s * ROWS_PER_TILE

    def previdx(t, gb, sb, semi):
        pltpu.async_copy(idx2.at[gdim, wid, pl.ds(t * CHUNK, CHUNK)],
                         gb, semi)
        pltpu.async_copy(idx2.at[sdim, wid, pl.ds(t * CHUNK, CHUNK)],
                         sb, semi)

    def waitidx(gb, sb, semi):
        pltpu.make_async_copy(idx2.at[gdim, wid, pl.ds(0, CHUNK)],
                              gb, semi).wait()
        pltpu.make_async_copy(idx2.at[sdim, wid, pl.ds(0, CHUNK)],
                              sb, semi).wait()

    previdx(0, gb0, sb0, semi0)
    previdx(1, gb1, sb1, semi1)

    zeros16 = jnp.zeros((LANES,), jnp.float32)
    def zrow(i, _):
        for j in range(CH // LANES):
            zbuf[i, pl.ds(j * LANES, LANES)] = zeros16
        return 0
    lax.fori_loop(0, ZROWS, zrow, 0)
    if with_hist:
        ones16 = jnp.full((LANES,), 1.0, jnp.float32)
        for j in range(CHUNK // LANES):
            ones[pl.ds(j * LANES, LANES)] = ones16
        for j in range(ROWS_PER_TILE // LANES):
            z640[pl.ds(j * LANES, LANES)] = zeros16

    for k in range(ZC):
        pltpu.sync_copy(zbuf, acc_sp.at[pl.ds(base + k * ZROWS, ZROWS)])
    if with_hist:
        pltpu.sync_copy(z640, hr_sp.at[pl.ds(base, ROWS_PER_TILE)])
        pltpu.sync_copy(z640, hc_sp.at[pl.ds(base, ROWS_PER_TILE)])
    plsc.subcore_barrier()

    def gather(gb, rows, sem):
        pltpu.async_copy(table.at[gb], rows, sem)

    def drain(rows, sem):
        pltpu.make_async_copy(table.at[pl.ds(0, CHUNK)], rows, sem).wait()

    def finish(rows, gb, sb):
        pltpu.sync_copy(rows, acc_sp.at[sb], add=True)
        if with_hist:
            pltpu.sync_copy(ones, hr_sp.at[gb], add=True)
            pltpu.sync_copy(ones, hc_sp.at[sb], add=True)

    waitidx(gb0, sb0, semi0)
    gather(gb0, rows0, sem0)
    def pair(j, _):
        t0 = 2 * j
        waitidx(gb1, sb1, semi1)
        gather(gb1, rows1, sem1)
        drain(rows0, sem0)
        finish(rows0, gb0, sb0)
        @pl.when(t0 + 2 < NCHUNK)
        def _():
            previdx(t0 + 2, gb0, sb0, semi0)
        drain(rows1, sem1)
        finish(rows1, gb1, sb1)
        @pl.when(t0 + 2 < NCHUNK)
        def _():
            waitidx(gb0, sb0, semi0)
            gather(gb0, rows0, sem0)
        @pl.when(t0 + 3 < NCHUNK)
        def _():
            previdx(t0 + 3, gb1, sb1, semi1)
        return 0
    lax.fori_loop(0, NCHUNK // 2, pair, 0)
    plsc.subcore_barrier()

    pltpu.sync_copy(acc_sp.at[pl.ds(base, ROWS_PER_TILE)],
                    p_out.at[c, pl.ds(base, ROWS_PER_TILE)])
    if with_hist:
        pltpu.sync_copy(hr_sp.at[pl.ds(base, ROWS_PER_TILE)],
                        hr_out.at[c, pl.ds(base, ROWS_PER_TILE)])
        pltpu.sync_copy(hc_sp.at[pl.ds(base, ROWS_PER_TILE)],
                        hc_out.at[c, pl.ds(base, ROWS_PER_TILE)])


def _sc_pass_a_body(idx2, table, hr_out, hc_out, p_out, *scratch):
    _seg_body(True, 0, 1, idx2, table, hr_out, hc_out, p_out, *scratch)


def _sc_pass_b_body(idx2, table, p_out, *scratch):
    _seg_body(False, 1, 0, idx2, table, p_out, *scratch)


@functools.lru_cache(maxsize=None)
def _sc_pass_a():
    return pl.kernel(
        _sc_pass_a_body,
        out_type=(jax.ShapeDtypeStruct((NC, NP), jnp.float32),
                  jax.ShapeDtypeStruct((NC, NP), jnp.float32),
                  jax.ShapeDtypeStruct((NC, NP, CH), jnp.float32)),
        mesh=_mesh(),
        scratch_types=(
            pltpu.VMEM((ZROWS, CH), jnp.float32),
            pltpu.VMEM((ROWS_PER_TILE,), jnp.float32),
            pltpu.VMEM((CHUNK,), jnp.float32),
            pltpu.VMEM((CHUNK,), jnp.int32),
            pltpu.VMEM((CHUNK,), jnp.int32),
            pltpu.VMEM((CHUNK,), jnp.int32),
            pltpu.VMEM((CHUNK,), jnp.int32),
            pltpu.VMEM((CHUNK, CH), jnp.float32),
            pltpu.VMEM((CHUNK, CH), jnp.float32),
            pltpu.VMEM_SHARED((NP,), jnp.float32),
            pltpu.VMEM_SHARED((NP,), jnp.float32),
            pltpu.VMEM_SHARED((NP, CH), jnp.float32),
            pltpu.SemaphoreType.DMA,
            pltpu.SemaphoreType.DMA,
            pltpu.SemaphoreType.DMA,
            pltpu.SemaphoreType.DMA,
        ),
    )


@functools.lru_cache(maxsize=None)
def _sc_pass_b():
    return pl.kernel(
        _sc_pass_b_body,
        out_type=jax.ShapeDtypeStruct((NC, NP, CH), jnp.float32),
        mesh=_mesh(),
        scratch_types=(
            pltpu.VMEM((ZROWS, CH), jnp.float32),
            pltpu.VMEM((CHUNK,), jnp.int32),
            pltpu.VMEM((CHUNK,), jnp.int32),
            pltpu.VMEM((CHUNK,), jnp.int32),
            pltpu.VMEM((CHUNK,), jnp.int32),
            pltpu.VMEM((CHUNK, CH), jnp.float32),
            pltpu.VMEM((CHUNK, CH), jnp.float32),
            pltpu.VMEM_SHARED((NP, CH), jnp.float32),
            pltpu.SemaphoreType.DMA,
            pltpu.SemaphoreType.DMA,
            pltpu.SemaphoreType.DMA,
            pltpu.SemaphoreType.DMA,
        ),
    )


_BLK = 1024


def _combine_body(h0_ref, h1_ref, p0_ref, p1_ref, o_ref):
    b = h0_ref[...] + h1_ref[...]
    scale = jnp.where(b > 0, 1.0 / b, 0.0)
    o_ref[...] = scale * (p0_ref[...] + p1_ref[...])


_combine = pl.pallas_call(
    _combine_body,
    grid=(NP // _BLK,),
    in_specs=[
        pl.BlockSpec((_BLK, 1), lambda i: (i, 0)),
        pl.BlockSpec((_BLK, 1), lambda i: (i, 0)),
        pl.BlockSpec((_BLK, CH), lambda i: (i, 0)),
        pl.BlockSpec((_BLK, CH), lambda i: (i, 0)),
    ],
    out_specs=pl.BlockSpec((_BLK, CH), lambda i: (i, 0)),
    out_shape=jax.ShapeDtypeStruct((NP, CH), jnp.float32),
)


def _final_body(h0_ref, h1_ref, q0_ref, q1_ref, w_ref, b_ref, o_ref):
    d = h0_ref[...] + h1_ref[...]
    scale = jnp.where(d > 0, 1.0 / d, 0.0)
    y = scale * (q0_ref[...] + q1_ref[...])
    o_ref[...] = lax.dot_general(
        y, w_ref[...], (((1,), (1,)), ((), ())),
        preferred_element_type=jnp.float32) + b_ref[...]


_final = pl.pallas_call(
    _final_body,
    grid=(NP // _BLK,),
    in_specs=[
        pl.BlockSpec((_BLK, 1), lambda i: (i, 0)),
        pl.BlockSpec((_BLK, 1), lambda i: (i, 0)),
        pl.BlockSpec((_BLK, CH), lambda i: (i, 0)),
        pl.BlockSpec((_BLK, CH), lambda i: (i, 0)),
        pl.BlockSpec((CH, CH), lambda i: (0, 0)),
        pl.BlockSpec((1, CH), lambda i: (0, 0)),
    ],
    out_specs=pl.BlockSpec((_BLK, CH), lambda i: (i, 0)),
    out_shape=jax.ShapeDtypeStruct((NP, CH), jnp.float32),
)


def kernel(x, hyperedge_index, W, bias):
    fill = DUMMY + jnp.arange(NNZ_P - NNZ, dtype=jnp.int32) % (NP - N_NODES)
    idx2 = jnp.stack([jnp.concatenate([hyperedge_index[0], fill]),
                      jnp.concatenate([hyperedge_index[1], fill])])
    idx2 = idx2.reshape(2, NW, EPT)
    x_pad = jnp.concatenate(
        [x, jnp.zeros((NP - N_NODES, CH), jnp.float32)])

    hr, hc, p = _sc_pass_a()(idx2, x_pad)
    e_pad = _combine(hc[0].reshape(NP, 1), hc[1].reshape(NP, 1), p[0], p[1])
    q = _sc_pass_b()(idx2, e_pad)
    out_pad = _final(hr[0].reshape(NP, 1), hr[1].reshape(NP, 1),
                     q[0], q[1], W, bias.reshape(1, CH))
    return out_pad[:N_NODES]


# speedup vs baseline: 31.3033x; 1.1038x over previous
import functools

import jax
import jax.numpy as jnp
from jax import lax
from jax.experimental import pallas as pl
from jax.experimental.pallas import tpu as pltpu
from jax.experimental.pallas import tpu_sc as plsc

N_NODES = 10000
NNZ = 320000
CH = 128
NC, NS, LANES = 2, 16, 16
NW = NC * NS
NP = 10240
CHUNK = 128
NNZ_P = 327680
EPT = NNZ_P // NW
NCHUNK = EPT // CHUNK
ROWS_PER_TILE = NP // NS
ZROWS = 64
ZC = ROWS_PER_TILE // ZROWS
DUMMY = N_NODES


@functools.lru_cache(maxsize=None)
def _mesh():
    return plsc.VectorSubcoreMesh(core_axis_name="c", subcore_axis_name="s",
                                  num_cores=NC, num_subcores=NS)


def _seg_body(with_hist, gdim, sdim, idx2, table, *refs):
    if with_hist:
        (hr0_out, hr1_out, hc0_out, hc1_out, p0_out, p1_out,
         zbuf, z640, ones, gb0, gb1, sb0, sb1,
         rows0, rows1, hr_sp, hc_sp, acc_sp,
         semi0, semi1, sem0, sem1) = refs
    else:
        (p0_out, p1_out, zbuf, gb0, gb1, sb0, sb1, rows0, rows1,
         acc_sp, semi0, semi1, sem0, sem1) = refs

    c = lax.axis_index("c")
    s = lax.axis_index("s")
    wid = c * NS + s
    base = s * ROWS_PER_TILE
    nch = jnp.where(wid < 2, 80, 78)
    cbase = 78 * wid + 2 * jnp.minimum(wid, 2)

    def previdx(t, gb, sb, semi):
        off = (cbase + t) * CHUNK
        pltpu.async_copy(idx2.at[gdim, pl.ds(off, CHUNK)], gb, semi)
        pltpu.async_copy(idx2.at[sdim, pl.ds(off, CHUNK)], sb, semi)

    def waitidx(gb, sb, semi):
        pltpu.make_async_copy(idx2.at[gdim, pl.ds(0, CHUNK)],
                              gb, semi).wait()
        pltpu.make_async_copy(idx2.at[sdim, pl.ds(0, CHUNK)],
                              sb, semi).wait()

    previdx(0, gb0, sb0, semi0)
    previdx(1, gb1, sb1, semi1)

    zeros16 = jnp.zeros((LANES,), jnp.float32)
    def zrow(i, _):
        for j in range(CH // LANES):
            zbuf[i, pl.ds(j * LANES, LANES)] = zeros16
        return 0
    lax.fori_loop(0, ZROWS, zrow, 0)
    if with_hist:
        ones16 = jnp.full((LANES,), 1.0, jnp.float32)
        for j in range(CHUNK // LANES):
            ones[pl.ds(j * LANES, LANES)] = ones16
        for j in range(ROWS_PER_TILE // LANES):
            z640[pl.ds(j * LANES, LANES)] = zeros16

    for k in range(ZC):
        pltpu.sync_copy(zbuf, acc_sp.at[pl.ds(base + k * ZROWS, ZROWS)])
    if with_hist:
        pltpu.sync_copy(z640, hr_sp.at[pl.ds(base, ROWS_PER_TILE)])
        pltpu.sync_copy(z640, hc_sp.at[pl.ds(base, ROWS_PER_TILE)])
    plsc.subcore_barrier()

    def gather(gb, rows, sem):
        pltpu.async_copy(table.at[gb], rows, sem)

    def drain(rows, sem):
        pltpu.make_async_copy(table.at[pl.ds(0, CHUNK)], rows, sem).wait()

    def finish(rows, gb, sb):
        pltpu.sync_copy(rows, acc_sp.at[sb], add=True)
        if with_hist:
            pltpu.sync_copy(ones, hr_sp.at[gb], add=True)
            pltpu.sync_copy(ones, hc_sp.at[sb], add=True)

    waitidx(gb0, sb0, semi0)
    gather(gb0, rows0, sem0)
    def pair(j, _):
        t0 = 2 * j
        waitidx(gb1, sb1, semi1)
        gather(gb1, rows1, sem1)
        drain(rows0, sem0)
        finish(rows0, gb0, sb0)
        @pl.when(t0 + 2 < nch)
        def _():
            previdx(t0 + 2, gb0, sb0, semi0)
        drain(rows1, sem1)
        finish(rows1, gb1, sb1)
        @pl.when(t0 + 2 < nch)
        def _():
            waitidx(gb0, sb0, semi0)
            gather(gb0, rows0, sem0)
        @pl.when(t0 + 3 < nch)
        def _():
            previdx(t0 + 3, gb1, sb1, semi1)
        return 0
    lax.fori_loop(0, nch // 2, pair, 0)
    plsc.subcore_barrier()

    sl = pl.ds(base, ROWS_PER_TILE)
    @pl.when(c == 0)
    def _():
        pltpu.sync_copy(acc_sp.at[sl], p0_out.at[sl])
        if with_hist:
            pltpu.sync_copy(hr_sp.at[sl], hr0_out.at[sl])
            pltpu.sync_copy(hc_sp.at[sl], hc0_out.at[sl])
    @pl.when(c == 1)
    def _():
        pltpu.sync_copy(acc_sp.at[sl], p1_out.at[sl])
        if with_hist:
            pltpu.sync_copy(hr_sp.at[sl], hr1_out.at[sl])
            pltpu.sync_copy(hc_sp.at[sl], hc1_out.at[sl])


def _sc_pass_a_body(idx2, table, hr0, hr1, hc0, hc1, p0, p1, *scratch):
    _seg_body(True, 0, 1, idx2, table, hr0, hr1, hc0, hc1, p0, p1, *scratch)


def _sc_pass_b_body(idx2, table, p0, p1, *scratch):
    _seg_body(False, 1, 0, idx2, table, p0, p1, *scratch)


@functools.lru_cache(maxsize=None)
def _sc_pass_a():
    return pl.kernel(
        _sc_pass_a_body,
        out_type=(jax.ShapeDtypeStruct((NP,), jnp.float32),
                  jax.ShapeDtypeStruct((NP,), jnp.float32),
                  jax.ShapeDtypeStruct((NP,), jnp.float32),
                  jax.ShapeDtypeStruct((NP,), jnp.float32),
                  jax.ShapeDtypeStruct((NP, CH), jnp.float32),
                  jax.ShapeDtypeStruct((NP, CH), jnp.float32)),
        mesh=_mesh(),
        scratch_types=(
            pltpu.VMEM((ZROWS, CH), jnp.float32),
            pltpu.VMEM((ROWS_PER_TILE,), jnp.float32),
            pltpu.VMEM((CHUNK,), jnp.float32),
            pltpu.VMEM((CHUNK,), jnp.int32),
            pltpu.VMEM((CHUNK,), jnp.int32),
            pltpu.VMEM((CHUNK,), jnp.int32),
            pltpu.VMEM((CHUNK,), jnp.int32),
            pltpu.VMEM((CHUNK, CH), jnp.float32),
            pltpu.VMEM((CHUNK, CH), jnp.float32),
            pltpu.VMEM_SHARED((NP,), jnp.float32),
            pltpu.VMEM_SHARED((NP,), jnp.float32),
            pltpu.VMEM_SHARED((NP, CH), jnp.float32),
            pltpu.SemaphoreType.DMA,
            pltpu.SemaphoreType.DMA,
            pltpu.SemaphoreType.DMA,
            pltpu.SemaphoreType.DMA,
        ),
    )


@functools.lru_cache(maxsize=None)
def _sc_pass_b():
    return pl.kernel(
        _sc_pass_b_body,
        out_type=(jax.ShapeDtypeStruct((NP, CH), jnp.float32),
                  jax.ShapeDtypeStruct((NP, CH), jnp.float32)),
        mesh=_mesh(),
        scratch_types=(
            pltpu.VMEM((ZROWS, CH), jnp.float32),
            pltpu.VMEM((CHUNK,), jnp.int32),
            pltpu.VMEM((CHUNK,), jnp.int32),
            pltpu.VMEM((CHUNK,), jnp.int32),
            pltpu.VMEM((CHUNK,), jnp.int32),
            pltpu.VMEM((CHUNK, CH), jnp.float32),
            pltpu.VMEM((CHUNK, CH), jnp.float32),
            pltpu.VMEM_SHARED((NP, CH), jnp.float32),
            pltpu.SemaphoreType.DMA,
            pltpu.SemaphoreType.DMA,
            pltpu.SemaphoreType.DMA,
            pltpu.SemaphoreType.DMA,
        ),
    )


_BLK = 1024


def _combine_body(h0_ref, h1_ref, p0_ref, p1_ref, o_ref):
    b = h0_ref[...] + h1_ref[...]
    scale = jnp.where(b > 0, 1.0 / b, 0.0)
    o_ref[...] = scale * (p0_ref[...] + p1_ref[...])


_combine = pl.pallas_call(
    _combine_body,
    grid=(NP // _BLK,),
    in_specs=[
        pl.BlockSpec((_BLK, 1), lambda i: (i, 0)),
        pl.BlockSpec((_BLK, 1), lambda i: (i, 0)),
        pl.BlockSpec((_BLK, CH), lambda i: (i, 0)),
        pl.BlockSpec((_BLK, CH), lambda i: (i, 0)),
    ],
    out_specs=pl.BlockSpec((_BLK, CH), lambda i: (i, 0)),
    out_shape=jax.ShapeDtypeStruct((NP, CH), jnp.float32),
)


def _final_body(h0_ref, h1_ref, q0_ref, q1_ref, w_ref, b_ref, o_ref):
    d = h0_ref[...] + h1_ref[...]
    scale = jnp.where(d > 0, 1.0 / d, 0.0)
    y = scale * (q0_ref[...] + q1_ref[...])
    o_ref[...] = lax.dot_general(
        y, w_ref[...], (((1,), (1,)), ((), ())),
        preferred_element_type=jnp.float32) + b_ref[...]


_final = pl.pallas_call(
    _final_body,
    grid=(NP // _BLK,),
    in_specs=[
        pl.BlockSpec((_BLK, 1), lambda i: (i, 0)),
        pl.BlockSpec((_BLK, 1), lambda i: (i, 0)),
        pl.BlockSpec((_BLK, CH), lambda i: (i, 0)),
        pl.BlockSpec((_BLK, CH), lambda i: (i, 0)),
        pl.BlockSpec((CH, CH), lambda i: (0, 0)),
        pl.BlockSpec((1, CH), lambda i: (0, 0)),
    ],
    out_specs=pl.BlockSpec((_BLK, CH), lambda i: (i, 0)),
    out_shape=jax.ShapeDtypeStruct((NP, CH), jnp.float32),
)


def kernel(x, hyperedge_index, W, bias):
    hr0, hr1, hc0, hc1, p0, p1 = _sc_pass_a()(hyperedge_index, x)
    e_pad = _combine(hc0.reshape(NP, 1), hc1.reshape(NP, 1), p0, p1)
    q0, q1 = _sc_pass_b()(hyperedge_index, e_pad)
    out_pad = _final(hr0.reshape(NP, 1), hr1.reshape(NP, 1),
                     q0, q1, W, bias.reshape(1, CH))
    return out_pad[:N_NODES]


# speedup vs baseline: 36.9860x; 1.1815x over previous
import functools

import jax
import jax.numpy as jnp
from jax import lax
from jax.experimental import pallas as pl
from jax.experimental.pallas import tpu as pltpu
from jax.experimental.pallas import tpu_sc as plsc

N_NODES = 10000
NNZ = 320000
CH = 128
HC = CH // 2
NC, NS, LANES = 2, 16, 16
NP = 10240
CHUNK = 128
NCH_LO = 156
NCH_HI = 160
ROWS_PER_TILE = NP // NS


@functools.lru_cache(maxsize=None)
def _mesh():
    return plsc.VectorSubcoreMesh(core_axis_name="c", subcore_axis_name="s",
                                  num_cores=NC, num_subcores=NS)


def _seg_body(with_hist, gdim, sdim, idx2, tab0, tab1, *refs):
    if with_hist:
        (hr_out, hc_out, p0_out, p1_out, zbuf, z640, ones,
         gall, sall, r0, r1, r2, r3, hist_sp, acc_sp,
         g0, g1, g2, g3, s0, s1, s2, s3) = refs
    else:
        (p0_out, p1_out, zbuf, gall, sall, r0, r1, r2, r3, acc_sp,
         g0, g1, g2, g3, s0, s1, s2, s3) = refs
    rows = (r0, r1, r2, r3)
    sem_g = (g0, g1, g2, g3)
    sem_s = (s0, s1, s2, s3)

    c = lax.axis_index("c")
    s = lax.axis_index("s")
    base = s * ROWS_PER_TILE
    nch = jnp.where(s == 0, NCH_HI, NCH_LO)
    cbase = NCH_LO * s + (NCH_HI - NCH_LO) * jnp.minimum(s, 1)
    ext = (NCH_HI - NCH_LO) * CHUNK

    pltpu.async_copy(idx2.at[gdim, pl.ds(cbase * CHUNK, NCH_LO * CHUNK)],
                     gall.at[pl.ds(0, NCH_LO * CHUNK)], sem_g[0])
    pltpu.async_copy(idx2.at[sdim, pl.ds(cbase * CHUNK, NCH_LO * CHUNK)],
                     sall.at[pl.ds(0, NCH_LO * CHUNK)], sem_g[1])
    @pl.when(s == 0)
    def _():
        pltpu.async_copy(idx2.at[gdim, pl.ds(NCH_LO * CHUNK, ext)],
                         gall.at[pl.ds(NCH_LO * CHUNK, ext)], sem_g[2])
        pltpu.async_copy(idx2.at[sdim, pl.ds(NCH_LO * CHUNK, ext)],
                         sall.at[pl.ds(NCH_LO * CHUNK, ext)], sem_g[3])

    zeros16 = jnp.zeros((LANES,), jnp.float32)
    def zrow(i, _):
        for j in range(HC // LANES):
            zbuf[i, pl.ds(j * LANES, LANES)] = zeros16
        return 0
    lax.fori_loop(0, CHUNK, zrow, 0)
    if with_hist:
        ones16 = jnp.full((LANES,), 1.0, jnp.float32)
        for j in range(CHUNK // LANES):
            ones[pl.ds(j * LANES, LANES)] = ones16
        for j in range(ROWS_PER_TILE // LANES):
            z640[pl.ds(j * LANES, LANES)] = zeros16

    for k in range(ROWS_PER_TILE // CHUNK):
        pltpu.sync_copy(zbuf, acc_sp.at[pl.ds(base + k * CHUNK, CHUNK)])
    if with_hist:
        pltpu.sync_copy(z640, hist_sp.at[pl.ds(base, ROWS_PER_TILE)])

    pltpu.make_async_copy(idx2.at[gdim, pl.ds(0, NCH_LO * CHUNK)],
                          gall.at[pl.ds(0, NCH_LO * CHUNK)], sem_g[0]).wait()
    pltpu.make_async_copy(idx2.at[sdim, pl.ds(0, NCH_LO * CHUNK)],
                          sall.at[pl.ds(0, NCH_LO * CHUNK)], sem_g[1]).wait()
    @pl.when(s == 0)
    def _():
        pltpu.make_async_copy(idx2.at[gdim, pl.ds(0, ext)],
                              gall.at[pl.ds(0, ext)], sem_g[2]).wait()
        pltpu.make_async_copy(idx2.at[sdim, pl.ds(0, ext)],
                              sall.at[pl.ds(0, ext)], sem_g[3]).wait()
    plsc.subcore_barrier()

    def gather(t, u):
        idxsl = gall.at[pl.ds(t * CHUNK, CHUNK)]
        @pl.when(c == 0)
        def _():
            pltpu.async_copy(tab0.at[idxsl], rows[u], sem_g[u])
        @pl.when(c == 1)
        def _():
            pltpu.async_copy(tab1.at[idxsl], rows[u], sem_g[u])

    def scat(t, u):
        pltpu.make_async_copy(tab0.at[pl.ds(0, CHUNK)],
                              rows[u], sem_g[u]).wait()
        pltpu.async_copy(rows[u], acc_sp.at[sall.at[pl.ds(t * CHUNK, CHUNK)]],
                         sem_s[u], add=True)
        if with_hist:
            @pl.when(c == 0)
            def _():
                pltpu.async_copy(ones,
                                 hist_sp.at[gall.at[pl.ds(t * CHUNK, CHUNK)]],
                                 sem_s[u], add=True)
            @pl.when(c == 1)
            def _():
                pltpu.async_copy(ones,
                                 hist_sp.at[sall.at[pl.ds(t * CHUNK, CHUNK)]],
                                 sem_s[u], add=True)

    def drain_scat(u):
        pltpu.make_async_copy(rows[u], acc_sp.at[pl.ds(0, CHUNK)],
                              sem_s[u]).wait()
        if with_hist:
            pltpu.make_async_copy(ones, hist_sp.at[pl.ds(0, CHUNK)],
                                  sem_s[u]).wait()

    gather(0, 0)
    gather(1, 1)
    def quad(j, _):
        for u in range(4):
            t = 4 * j + u
            @pl.when(t >= 2)
            def _(u=u):
                drain_scat((u + 2) % 4)
            @pl.when(t + 2 < nch)
            def _(t=t, u=u):
                gather(t + 2, (u + 2) % 4)
            scat(t, u)
        return 0
    lax.fori_loop(0, nch // 4, quad, 0)
    drain_scat(2)
    drain_scat(3)
    plsc.subcore_barrier()

    sl = pl.ds(base, ROWS_PER_TILE)
    @pl.when(c == 0)
    def _():
        pltpu.sync_copy(acc_sp.at[sl], p0_out.at[sl])
        if with_hist:
            pltpu.sync_copy(hist_sp.at[sl], hr_out.at[sl])
    @pl.when(c == 1)
    def _():
        pltpu.sync_copy(acc_sp.at[sl], p1_out.at[sl])
        if with_hist:
            pltpu.sync_copy(hist_sp.at[sl], hc_out.at[sl])


def _sc_pass_a_body(idx2, tab0, tab1, hr, hc, p0, p1, *scratch):
    _seg_body(True, 0, 1, idx2, tab0, tab1, hr, hc, p0, p1, *scratch)


def _sc_pass_b_body(idx2, tab0, tab1, p0, p1, *scratch):
    _seg_body(False, 1, 0, idx2, tab0, tab1, p0, p1, *scratch)


_SEG_SCRATCH_TAIL = (
    pltpu.VMEM((NCH_HI * CHUNK,), jnp.int32),
    pltpu.VMEM((NCH_HI * CHUNK,), jnp.int32),
    pltpu.VMEM((CHUNK, HC), jnp.float32),
    pltpu.VMEM((CHUNK, HC), jnp.float32),
    pltpu.VMEM((CHUNK, HC), jnp.float32),
    pltpu.VMEM((CHUNK, HC), jnp.float32),
)
_SEG_SEMS = (pltpu.SemaphoreType.DMA,) * 8


@functools.lru_cache(maxsize=None)
def _sc_pass_a():
    return pl.kernel(
        _sc_pass_a_body,
        out_type=(jax.ShapeDtypeStruct((NP,), jnp.float32),
                  jax.ShapeDtypeStruct((NP,), jnp.float32),
                  jax.ShapeDtypeStruct((NP, HC), jnp.float32),
                  jax.ShapeDtypeStruct((NP, HC), jnp.float32)),
        mesh=_mesh(),
        compiler_params=pltpu.CompilerParams(use_tc_tiling_on_sc=False),
        scratch_types=(
            pltpu.VMEM((CHUNK, HC), jnp.float32),
            pltpu.VMEM((ROWS_PER_TILE,), jnp.float32),
            pltpu.VMEM((CHUNK,), jnp.float32),
        ) + _SEG_SCRATCH_TAIL + (
            pltpu.VMEM_SHARED((NP,), jnp.float32),
            pltpu.VMEM_SHARED((NP, HC), jnp.float32),
        ) + _SEG_SEMS,
    )


@functools.lru_cache(maxsize=None)
def _sc_pass_b():
    return pl.kernel(
        _sc_pass_b_body,
        out_type=(jax.ShapeDtypeStruct((NP, HC), jnp.float32),
                  jax.ShapeDtypeStruct((NP, HC), jnp.float32)),
        mesh=_mesh(),
        compiler_params=pltpu.CompilerParams(use_tc_tiling_on_sc=False),
        scratch_types=(
            pltpu.VMEM((CHUNK, HC), jnp.float32),
        ) + _SEG_SCRATCH_TAIL + (
            pltpu.VMEM_SHARED((NP, HC), jnp.float32),
        ) + _SEG_SEMS,
    )


_BLK = 1024


def _combine_body(h_ref, p0_ref, p1_ref, e0_ref, e1_ref):
    b = h_ref[...]
    scale = jnp.where(b > 0, 1.0 / b, 0.0)
    e0_ref[...] = scale * p0_ref[...]
    e1_ref[...] = scale * p1_ref[...]


_combine = pl.pallas_call(
    _combine_body,
    grid=(NP // _BLK,),
    in_specs=[
        pl.BlockSpec((_BLK, 1), lambda i: (i, 0)),
        pl.BlockSpec((_BLK, HC), lambda i: (i, 0)),
        pl.BlockSpec((_BLK, HC), lambda i: (i, 0)),
    ],
    out_specs=[
        pl.BlockSpec((_BLK, HC), lambda i: (i, 0)),
        pl.BlockSpec((_BLK, HC), lambda i: (i, 0)),
    ],
    out_shape=(jax.ShapeDtypeStruct((NP, HC), jnp.float32),
               jax.ShapeDtypeStruct((NP, HC), jnp.float32)),
)


def _final_body(h_ref, q0_ref, q1_ref, w_ref, b_ref, o_ref):
    d = h_ref[...]
    scale = jnp.where(d > 0, 1.0 / d, 0.0)
    y = jnp.concatenate([scale * q0_ref[...], scale * q1_ref[...]], axis=1)
    o_ref[...] = lax.dot_general(
        y, w_ref[...], (((1,), (1,)), ((), ())),
        preferred_element_type=jnp.float32) + b_ref[...]


_final = pl.pallas_call(
    _final_body,
    grid=(NP // _BLK,),
    in_specs=[
        pl.BlockSpec((_BLK, 1), lambda i: (i, 0)),
        pl.BlockSpec((_BLK, HC), lambda i: (i, 0)),
        pl.BlockSpec((_BLK, HC), lambda i: (i, 0)),
        pl.BlockSpec((CH, CH), lambda i: (0, 0)),
        pl.BlockSpec((1, CH), lambda i: (0, 0)),
    ],
    out_specs=pl.BlockSpec((_BLK, CH), lambda i: (i, 0)),
    out_shape=jax.ShapeDtypeStruct((NP, CH), jnp.float32),
)


def kernel(x, hyperedge_index, W, bias):
    x0 = x[:, :HC]
    x1 = x[:, HC:]
    hr, hc, p0, p1 = _sc_pass_a()(hyperedge_index, x0, x1)
    e0, e1 = _combine(hc.reshape(NP, 1), p0, p1)
    q0, q1 = _sc_pass_b()(hyperedge_index, e0, e1)
    out_pad = _final(hr.reshape(NP, 1), q0, q1, W, bias.reshape(1, CH))
    return out_pad[:N_NODES]
